# baseline (speedup 1.0000x reference)
import math

import jax
import jax.numpy as jnp
from jax import lax
from jax.experimental import pallas as pl
from jax.experimental.pallas import tpu as pltpu
from jax.experimental.pallas import tpu_sc as plsc

_D = 64
_DP = 128
_SCALE = math.sqrt(_D)
_NC, _NS = 2, 16
_NW = _NC * _NS
_NR = 4


def _transpose(v):
    n_full = v // _DP
    jobs_per = (n_full + _NW - 1) // _NW
    mesh = plsc.VectorSubcoreMesh(
        core_axis_name="c", subcore_axis_name="s",
        num_cores=_NC, num_subcores=_NS)

    def body(tt_hbm, tail_hbm, out_hbm, src0, src1, dst0, dst1,
             isem0, isem1, wsem0, wsem1):
        wid = lax.axis_index("s") * _NC + lax.axis_index("c")
        iotas = [lax.iota(jnp.int32, 16) + 16 * f8 for f8 in range(4)]

        def col0(k):
            j = jnp.minimum(wid + _NW * k, n_full - 1)
            return pl.multiple_of(j * _DP, _DP)

        def fire(src_v, isem, k):
            pltpu.async_copy(tt_hbm.at[:, pl.ds(col0(k), _DP)], src_v, isem)

        def compute(src_v, dst_v, wsem, k):
            def vloop(vv, c):
                cols = jnp.full((16,), vv, jnp.int32)
                for f8 in range(4):
                    x = plsc.load_gather(src_v, [iotas[f8], cols])
                    dst_v[vv, pl.ds(16 * f8, 16)] = x * _SCALE
                return c

            lax.fori_loop(0, _DP, vloop, 0)
            pltpu.async_copy(dst_v, out_hbm.at[pl.ds(col0(k), _DP)], wsem)

        def drain_in(src_v, isem):
            pltpu.make_async_copy(tt_hbm.at[:, pl.ds(0, _DP)], src_v,
                                  isem).wait()

        def drain_out(dst_v, wsem):
            pltpu.make_async_copy(dst_v, out_hbm.at[pl.ds(0, _DP)],
                                  wsem).wait()

        bufs = [(src0, dst0, isem0, wsem0), (src1, dst1, isem1, wsem1)]
        fire(src0, isem0, 0)
        drain_in(src0, isem0)
        fire(src1, isem1, 1)
        compute(src0, dst0, wsem0, 0)
        drain_in(src1, isem1)
        fire(src0, isem0, 2)
        compute(src1, dst1, wsem1, 1)

        n_pairs = max(0, (jobs_per - 3) // 2)

        def loop(g2, carry):
            for p in range(2):
                k = 2 * g2 + 2 + p
                s_v, d_v, isem, wsem = bufs[p]
                drain_in(s_v, isem)
                drain_out(d_v, wsem)
                fire(bufs[1 - p][0], bufs[1 - p][2], k + 1)
                compute(s_v, d_v, wsem, k)
            return carry

        lax.fori_loop(0, n_pairs, loop, 0)
        for k in range(2 + 2 * n_pairs, jobs_per):
            s_v, d_v, isem, wsem = bufs[k % 2]
            drain_in(s_v, isem)
            drain_out(d_v, wsem)
            if k + 1 < jobs_per:
                fire(bufs[(k + 1) % 2][0], bufs[(k + 1) % 2][2], k + 1)
            compute(s_v, d_v, wsem, k)
        drain_out(bufs[(jobs_per - 1) % 2][1], bufs[(jobs_per - 1) % 2][3])
        drain_out(bufs[(jobs_per - 2) % 2][1], bufs[(jobs_per - 2) % 2][3])

        if v % _DP:
            t0, tw = n_full * _DP, v - n_full * _DP
            pltpu.sync_copy(tail_hbm, out_hbm.at[pl.ds(t0, tw)])

    return pl.kernel(
        body,
        out_type=jax.ShapeDtypeStruct((v, _DP), jnp.float32),
        mesh=mesh,
        compiler_params=pltpu.CompilerParams(needs_layout_passes=False),
        scratch_types=[
            pltpu.VMEM((_D, _DP), jnp.float32),
            pltpu.VMEM((_D, _DP), jnp.float32),
            pltpu.VMEM((_DP, _DP), jnp.float32),
            pltpu.VMEM((_DP, _DP), jnp.float32),
            pltpu.SemaphoreType.DMA,
            pltpu.SemaphoreType.DMA,
            pltpu.SemaphoreType.DMA,
            pltpu.SemaphoreType.DMA,
        ],
    )


def _lookup(batch, seq):
    bpw = batch // _NW
    seq_lo = min(seq, 128)
    seq_hi = seq - seq_lo
    mesh = plsc.VectorSubcoreMesh(
        core_axis_name="c", subcore_axis_name="s",
        num_cores=_NC, num_subcores=_NS)

    def body(idx_hbm, table_hbm, out_hbm, *refs):
        idx_v = refs[0:_NR]
        pad_v = refs[_NR:2 * _NR]
        gsem = refs[2 * _NR:3 * _NR]
        wsem = refs[3 * _NR:4 * _NR]
        wid = lax.axis_index("s") * _NC + lax.axis_index("c")
        b0 = wid * bpw

        def fire(j, bb):
            pltpu.sync_copy(idx_hbm.at[pl.ds(bb * seq, seq)], idx_v[j])
            pltpu.async_copy(
                table_hbm.at[idx_v[j].at[pl.ds(0, seq_lo)]],
                pad_v[j].at[pl.ds(0, seq_lo)], gsem[j])
            if seq_hi:
                pltpu.async_copy(
                    table_hbm.at[idx_v[j].at[pl.ds(seq_lo, seq_hi)]],
                    pad_v[j].at[pl.ds(seq_lo, seq_hi)], gsem[j])

        for j in range(2):
            fire(j, b0 + j)

        def step(j, g):
            bb = b0 + g
            nxt = (j + 2) % _NR

            @pl.when(g + 2 < bpw)
            def _():
                @pl.when(g >= 2)
                def _():
                    pltpu.make_async_copy(
                        pad_v[nxt], out_hbm.at[bb - 2], wsem[nxt]).wait()
                fire(nxt, bb + 2)

            pltpu.make_async_copy(out_hbm.at[bb], pad_v[j], gsem[j]).wait()
            pltpu.async_copy(pad_v[j], out_hbm.at[bb], wsem[j])

        def loop(g4, carry):
            for j in range(_NR):
                step(j, g4 * _NR + j)
            return carry

        lax.fori_loop(0, bpw // _NR, loop, 0)
        for j in range(_NR):
            pltpu.make_async_copy(pad_v[j], out_hbm.at[b0 + bpw - _NR + j],
                                  wsem[j]).wait()

    return pl.kernel(
        body,
        out_type=jax.ShapeDtypeStruct((batch, seq, _DP), jnp.float32),
        mesh=mesh,
        compiler_params=pltpu.CompilerParams(use_tc_tiling_on_sc=False),
        scratch_types=(
            [pltpu.VMEM((seq,), jnp.int32) for _ in range(_NR)]
            + [pltpu.VMEM((seq, _DP), jnp.float32) for _ in range(_NR)]
            + [pltpu.SemaphoreType.DMA for _ in range(2 * _NR)]
        ),
    )


def kernel(token_ids_batch, embeddings_table):
    b, s = token_ids_batch.shape
    v = embeddings_table.shape[0]
    idx = token_ids_batch.astype(jnp.int32).reshape(b * s)
    n_full = v // _DP
    tail_p = jnp.pad(embeddings_table[n_full * _DP:] * _SCALE,
                     ((0, 0), (0, _DP - _D)))
    table_p = _transpose(v)(embeddings_table.T, tail_p)
    out = _lookup(b, s)(idx, table_p)
    return out[:, :, :_D]


# speedup vs baseline: 1.1466x; 1.1466x over previous
import math

import jax
import jax.numpy as jnp
from jax import lax
from jax.experimental import pallas as pl
from jax.experimental.pallas import tpu as pltpu
from jax.experimental.pallas import tpu_sc as plsc

_D = 64
_DP = 128
_SCALE = math.sqrt(_D)
_NC, _NS = 2, 16
_NW = _NC * _NS
_NR = 4


def _transpose(v):
    n_full = v // _DP
    jobs_per = (n_full + _NW - 1) // _NW
    mesh = plsc.VectorSubcoreMesh(
        core_axis_name="c", subcore_axis_name="s",
        num_cores=_NC, num_subcores=_NS)

    def body(tt_hbm, tail_hbm, out_hbm, src0, src1, dst0, dst1,
             isem0, isem1, wsem0, wsem1):
        wid = lax.axis_index("s") * _NC + lax.axis_index("c")
        iotas8 = [lax.iota(jnp.int32, 16) + 16 * v8 for v8 in range(8)]

        def col0(k):
            j = jnp.minimum(wid + _NW * k, n_full - 1)
            return pl.multiple_of(j * _DP, _DP)

        def fire(src_v, isem, k):
            pltpu.async_copy(tt_hbm.at[:, pl.ds(col0(k), _DP)], src_v, isem)

        def compute(src_v, dst_v, wsem, k):
            def floop(f2, c):
                for u in range(2):
                    f = f2 * 2 + u
                    cols = jnp.full((16,), f, jnp.int32)
                    for v8 in range(8):
                        x = src_v[f, pl.ds(16 * v8, 16)]
                        plsc.store_scatter(dst_v, [iotas8[v8], cols],
                                           x * _SCALE)
                return c

            lax.fori_loop(0, _D // 2, floop, 0)
            pltpu.async_copy(dst_v, out_hbm.at[pl.ds(col0(k), _DP)], wsem)

        def drain_in(src_v, isem):
            pltpu.make_async_copy(tt_hbm.at[:, pl.ds(0, _DP)], src_v,
                                  isem).wait()

        def drain_out(dst_v, wsem):
            pltpu.make_async_copy(dst_v, out_hbm.at[pl.ds(0, _DP)],
                                  wsem).wait()

        bufs = [(src0, dst0, isem0, wsem0), (src1, dst1, isem1, wsem1)]
        fire(src0, isem0, 0)
        drain_in(src0, isem0)
        fire(src1, isem1, 1)
        compute(src0, dst0, wsem0, 0)
        drain_in(src1, isem1)
        fire(src0, isem0, 2)
        compute(src1, dst1, wsem1, 1)

        n_pairs = max(0, (jobs_per - 3) // 2)

        def loop(g2, carry):
            for p in range(2):
                k = 2 * g2 + 2 + p
                s_v, d_v, isem, wsem = bufs[p]
                drain_in(s_v, isem)
                drain_out(d_v, wsem)
                fire(bufs[1 - p][0], bufs[1 - p][2], k + 1)
                compute(s_v, d_v, wsem, k)
            return carry

        lax.fori_loop(0, n_pairs, loop, 0)
        for k in range(2 + 2 * n_pairs, jobs_per):
            s_v, d_v, isem, wsem = bufs[k % 2]
            drain_in(s_v, isem)
            drain_out(d_v, wsem)
            if k + 1 < jobs_per:
                fire(bufs[(k + 1) % 2][0], bufs[(k + 1) % 2][2], k + 1)
            compute(s_v, d_v, wsem, k)
        drain_out(bufs[(jobs_per - 1) % 2][1], bufs[(jobs_per - 1) % 2][3])
        drain_out(bufs[(jobs_per - 2) % 2][1], bufs[(jobs_per - 2) % 2][3])

        if v % _DP:
            t0, tw = n_full * _DP, v - n_full * _DP
            pltpu.sync_copy(tail_hbm, out_hbm.at[pl.ds(t0, tw)])

    return pl.kernel(
        body,
        out_type=jax.ShapeDtypeStruct((v, _DP), jnp.float32),
        mesh=mesh,
        compiler_params=pltpu.CompilerParams(needs_layout_passes=False),
        scratch_types=[
            pltpu.VMEM((_D, _DP), jnp.float32),
            pltpu.VMEM((_D, _DP), jnp.float32),
            pltpu.VMEM((_DP, _DP), jnp.float32),
            pltpu.VMEM((_DP, _DP), jnp.float32),
            pltpu.SemaphoreType.DMA,
            pltpu.SemaphoreType.DMA,
            pltpu.SemaphoreType.DMA,
            pltpu.SemaphoreType.DMA,
        ],
    )


def _lookup(batch, seq):
    bpw = batch // _NW
    seq_lo = min(seq, 128)
    seq_hi = seq - seq_lo
    mesh = plsc.VectorSubcoreMesh(
        core_axis_name="c", subcore_axis_name="s",
        num_cores=_NC, num_subcores=_NS)

    def body(idx_hbm, table_hbm, out_hbm, *refs):
        idx_v = refs[0:_NR]
        pad_v = refs[_NR:2 * _NR]
        gsem = refs[2 * _NR:3 * _NR]
        wsem = refs[3 * _NR:4 * _NR]
        wid = lax.axis_index("s") * _NC + lax.axis_index("c")
        b0 = wid * bpw

        def fire(j, bb):
            pltpu.sync_copy(idx_hbm.at[pl.ds(bb * seq, seq)], idx_v[j])
            pltpu.async_copy(
                table_hbm.at[idx_v[j].at[pl.ds(0, seq_lo)]],
                pad_v[j].at[pl.ds(0, seq_lo)], gsem[j])
            if seq_hi:
                pltpu.async_copy(
                    table_hbm.at[idx_v[j].at[pl.ds(seq_lo, seq_hi)]],
                    pad_v[j].at[pl.ds(seq_lo, seq_hi)], gsem[j])

        for j in range(2):
            fire(j, b0 + j)

        def step(j, g):
            bb = b0 + g
            nxt = (j + 2) % _NR

            @pl.when(g + 2 < bpw)
            def _():
                @pl.when(g >= 2)
                def _():
                    pltpu.make_async_copy(
                        pad_v[nxt], out_hbm.at[bb - 2], wsem[nxt]).wait()
                fire(nxt, bb + 2)

            pltpu.make_async_copy(out_hbm.at[bb], pad_v[j], gsem[j]).wait()
            pltpu.async_copy(pad_v[j], out_hbm.at[bb], wsem[j])

        def loop(g4, carry):
            for j in range(_NR):
                step(j, g4 * _NR + j)
            return carry

        lax.fori_loop(0, bpw // _NR, loop, 0)
        for j in range(_NR):
            pltpu.make_async_copy(pad_v[j], out_hbm.at[b0 + bpw - _NR + j],
                                  wsem[j]).wait()

    return pl.kernel(
        body,
        out_type=jax.ShapeDtypeStruct((batch, seq, _DP), jnp.float32),
        mesh=mesh,
        compiler_params=pltpu.CompilerParams(use_tc_tiling_on_sc=False),
        scratch_types=(
            [pltpu.VMEM((seq,), jnp.int32) for _ in range(_NR)]
            + [pltpu.VMEM((seq, _DP), jnp.float32) for _ in range(_NR)]
            + [pltpu.SemaphoreType.DMA for _ in range(2 * _NR)]
        ),
    )


def kernel(token_ids_batch, embeddings_table):
    b, s = token_ids_batch.shape
    v = embeddings_table.shape[0]
    idx = token_ids_batch.astype(jnp.int32).reshape(b * s)
    n_full = v // _DP
    tail_p = jnp.pad(embeddings_table[n_full * _DP:] * _SCALE,
                     ((0, 0), (0, _DP - _D)))
    table_p = _transpose(v)(embeddings_table.T, tail_p)
    out = _lookup(b, s)(idx, table_p)
    return out[:, :, :_D]


# speedup vs baseline: 1.1908x; 1.0386x over previous
import math

import jax
import jax.numpy as jnp
from jax import lax
from jax.experimental import pallas as pl
from jax.experimental.pallas import tpu as pltpu
from jax.experimental.pallas import tpu_sc as plsc

_D = 64
_DW = 96
_DP = 128
_SCALE = math.sqrt(_D)
_NC, _NS = 2, 16
_NW = _NC * _NS
_NR = 4
_NP = 2


def _build(batch, seq):
    bpw = batch // _NW
    seq_lo = min(seq, 128)
    seq_hi = seq - seq_lo
    mesh = plsc.VectorSubcoreMesh(
        core_axis_name="c", subcore_axis_name="s",
        num_cores=_NC, num_subcores=_NS)

    def body(idx_hbm, table_hbm, out_hbm, *refs):
        idx_v = refs[0:_NR]
        den_v = refs[_NR:2 * _NR]
        pad_v = refs[2 * _NR:2 * _NR + _NP]
        gsem = refs[2 * _NR + _NP:3 * _NR + _NP]
        wsem = refs[3 * _NR + _NP:3 * _NR + 2 * _NP]
        wid = lax.axis_index("s") * _NC + lax.axis_index("c")
        b0 = wid * bpw

        def fire(j, bb):
            pltpu.sync_copy(idx_hbm.at[pl.ds(bb * seq, seq)], idx_v[j])
            pltpu.async_copy(
                table_hbm.at[idx_v[j].at[pl.ds(0, seq_lo)]],
                den_v[j].at[pl.ds(0, seq_lo)], gsem[j])
            if seq_hi:
                pltpu.async_copy(
                    table_hbm.at[idx_v[j].at[pl.ds(seq_lo, seq_hi)]],
                    den_v[j].at[pl.ds(seq_lo, seq_hi)], gsem[j])

        for j in range(2):
            fire(j, b0 + j)

        def step(j, q, g):
            bb = b0 + g
            nxt = (j + 2) % _NR

            @pl.when(g + 2 < bpw)
            def _():
                fire(nxt, bb + 2)

            pltpu.make_async_copy(
                out_hbm.at[bb, :, pl.ds(0, _DW)], den_v[j], gsem[j]).wait()

            @pl.when(g >= _NP)
            def _():
                pltpu.make_async_copy(
                    pad_v[q], out_hbm.at[bb - _NP], wsem[q]).wait()

            def scale(s4, c):
                for u in range(4):
                    for k in range(_D // 16):
                        sl = pl.ds(k * 16, 16)
                        pad_v[q][s4 * 4 + u, sl] = (
                            den_v[j][s4 * 4 + u, sl] * _SCALE)
                return c

            lax.fori_loop(0, seq // 4, scale, 0)
            pltpu.async_copy(pad_v[q], out_hbm.at[bb], wsem[q])

        def loop(g4, carry):
            for j in range(_NR):
                step(j, j % _NP, g4 * _NR + j)
            return carry

        lax.fori_loop(0, bpw // _NR, loop, 0)
        for q in range(_NP):
            pltpu.make_async_copy(pad_v[q], out_hbm.at[b0 + bpw - _NP + q],
                                  wsem[q]).wait()

    return pl.kernel(
        body,
        out_type=jax.ShapeDtypeStruct((batch, seq, _DP), jnp.float32),
        mesh=mesh,
        compiler_params=pltpu.CompilerParams(use_tc_tiling_on_sc=False),
        scratch_types=(
            [pltpu.VMEM((seq,), jnp.int32) for _ in range(_NR)]
            + [pltpu.VMEM((seq, _DW), jnp.float32) for _ in range(_NR)]
            + [pltpu.VMEM((seq, _DP), jnp.float32) for _ in range(_NP)]
            + [pltpu.SemaphoreType.DMA for _ in range(_NR + _NP)]
        ),
    )


def kernel(token_ids_batch, embeddings_table):
    b, s = token_ids_batch.shape
    idx = token_ids_batch.astype(jnp.int32).reshape(b * s)
    table_p = jnp.pad(embeddings_table, ((0, 0), (0, _DW - _D)))
    out = _build(b, s)(idx, table_p)
    return out[:, :, :_D]


# speedup vs baseline: 2.1201x; 1.7803x over previous
import math

import jax
import jax.numpy as jnp
from jax import lax
from jax.experimental import pallas as pl
from jax.experimental.pallas import tpu as pltpu
from jax.experimental.pallas import tpu_sc as plsc

_D = 64
_DP = 128
_SCALE = math.sqrt(_D)
_NC, _NS = 2, 16
_NW = _NC * _NS
_NR = 4


def _build(batch, seq):
    bpw = batch // _NW
    seq_lo = min(seq, 128)
    seq_hi = seq - seq_lo
    mesh = plsc.VectorSubcoreMesh(
        core_axis_name="c", subcore_axis_name="s",
        num_cores=_NC, num_subcores=_NS)

    def body(idx_hbm, table_hbm, out_hbm, *refs):
        idx_v = refs[0:_NR]
        pad_v = refs[_NR:2 * _NR]
        gsem = refs[2 * _NR:3 * _NR]
        wsem = refs[3 * _NR:4 * _NR]
        wid = lax.axis_index("s") * _NC + lax.axis_index("c")
        b0 = wid * bpw

        def fire(j, bb):
            pltpu.sync_copy(idx_hbm.at[pl.ds(bb * seq, seq)], idx_v[j])
            pltpu.async_copy(
                table_hbm.at[idx_v[j].at[pl.ds(0, seq_lo)]],
                pad_v[j].at[pl.ds(0, seq_lo)], gsem[j])
            if seq_hi:
                pltpu.async_copy(
                    table_hbm.at[idx_v[j].at[pl.ds(seq_lo, seq_hi)]],
                    pad_v[j].at[pl.ds(seq_lo, seq_hi)], gsem[j])

        for j in range(2):
            fire(j, b0 + j)

        def step(j, g):
            bb = b0 + g
            nxt = (j + 2) % _NR

            @pl.when(g + 2 < bpw)
            def _():
                @pl.when(g >= 2)
                def _():
                    pltpu.make_async_copy(
                        pad_v[nxt], out_hbm.at[bb - 2], wsem[nxt]).wait()
                fire(nxt, bb + 2)

            pltpu.make_async_copy(out_hbm.at[bb], pad_v[j], gsem[j]).wait()

            def scale(s4, c):
                for u in range(4):
                    for k in range(_D // 16):
                        sl = pl.ds(k * 16, 16)
                        pad_v[j][s4 * 4 + u, sl] = (
                            pad_v[j][s4 * 4 + u, sl] * _SCALE)
                return c

            lax.fori_loop(0, seq // 4, scale, 0)
            pltpu.async_copy(pad_v[j], out_hbm.at[bb], wsem[j])

        def loop(g4, carry):
            for j in range(_NR):
                step(j, g4 * _NR + j)
            return carry

        lax.fori_loop(0, bpw // _NR, loop, 0)
        for j in range(_NR):
            pltpu.make_async_copy(pad_v[j], out_hbm.at[b0 + bpw - _NR + j],
                                  wsem[j]).wait()

    return pl.kernel(
        body,
        out_type=jax.ShapeDtypeStruct((batch, seq, _DP), jnp.float32),
        mesh=mesh,
        compiler_params=pltpu.CompilerParams(use_tc_tiling_on_sc=False),
        scratch_types=(
            [pltpu.VMEM((seq,), jnp.int32) for _ in range(_NR)]
            + [pltpu.VMEM((seq, _DP), jnp.float32) for _ in range(_NR)]
            + [pltpu.SemaphoreType.DMA for _ in range(2 * _NR)]
        ),
    )


def kernel(token_ids_batch, embeddings_table):
    b, s = token_ids_batch.shape
    idx = token_ids_batch.astype(jnp.int32).reshape(b * s)
    table_p = jnp.pad(embeddings_table, ((0, 0), (0, _DP - _D)))
    out = _build(b, s)(idx, table_p)
    return out[:, :, :_D]
